# Initial kernel scaffold; baseline (speedup 1.0000x reference)
import jax
import jax.numpy as jnp
from jax.experimental import pallas as pl


def kernel(kx, vx, k_cache, v_cache):
    raise NotImplementedError("write your pallas kernel here")


# speedup vs baseline: 8.0711x; 8.0711x over previous
import jax
import jax.numpy as jnp
from jax import lax
from jax.experimental import pallas as pl
from jax.experimental.pallas import tpu as pltpu
from jax.experimental.pallas import tpu_sc as plsc

_NUM_WORKERS = 32


def kernel(kx, vx, k_cache, v_cache):
    B, _, D = kx.shape
    total = B * D
    chunk = total // _NUM_WORKERS
    kx1 = kx.reshape(total)
    vx1 = vx.reshape(total)

    mesh = plsc.VectorSubcoreMesh(core_axis_name="c", subcore_axis_name="s")

    def body(kx_hbm, vx_hbm, ko_hbm, vo_hbm, kbuf, vbuf):
        c = lax.axis_index("c")
        s = lax.axis_index("s")
        wid = s * 2 + c
        base = wid * chunk
        pltpu.sync_copy(kx_hbm.at[pl.ds(base, chunk)], kbuf)
        pltpu.sync_copy(vx_hbm.at[pl.ds(base, chunk)], vbuf)
        pltpu.sync_copy(kbuf, ko_hbm.at[pl.ds(base, chunk)])
        pltpu.sync_copy(vbuf, vo_hbm.at[pl.ds(base, chunk)])

    out_k, out_v = pl.kernel(
        body,
        mesh=mesh,
        out_type=(
            jax.ShapeDtypeStruct((total,), kx.dtype),
            jax.ShapeDtypeStruct((total,), vx.dtype),
        ),
        scratch_types=[
            pltpu.VMEM((chunk,), jnp.float32),
            pltpu.VMEM((chunk,), jnp.float32),
        ],
    )(kx1, vx1)

    return (out_k.reshape(B, 1, D), out_v.reshape(B, 1, D))
